# baseline (speedup 1.0000x reference)
import functools

import jax
import jax.numpy as jnp
from jax import lax
from jax.experimental import pallas as pl
from jax.experimental.pallas import tpu as pltpu
from jax.experimental.pallas import tpu_sc as plsc

_D = 256
_S = 8192
_B = 4
_NC = 2
_NS = 16
_NW = _NC * _NS
_ROWS = _S // _NW
_CHUNK = 128

_mesh = plsc.VectorSubcoreMesh(core_axis_name="c", subcore_axis_name="s")


@functools.partial(
    pl.kernel,
    mesh=_mesh,
    out_type=jax.ShapeDtypeStruct((_B, _S, _D), jnp.float32),
    scratch_types=[
        pltpu.VMEM((_ROWS, _D), jnp.float32),
        pltpu.SemaphoreType.DMA,
        pltpu.SemaphoreType.DMA,
    ],
)
def _broadcast_rows(emb_hbm, out_hbm, buf, rsem, wsem):
    wid = lax.axis_index("s") * _NC + lax.axis_index("c")
    base = wid * _ROWS
    nchunks = _ROWS // _CHUNK
    reads = [
        pltpu.async_copy(
            emb_hbm.at[pl.ds(base + i * _CHUNK, _CHUNK)],
            buf.at[pl.ds(i * _CHUNK, _CHUNK)],
            rsem,
        )
        for i in range(nchunks)
    ]
    writes = []
    for i in range(nchunks):
        reads[i].wait()
        writes += [
            pltpu.async_copy(
                buf.at[pl.ds(i * _CHUNK, _CHUNK)],
                out_hbm.at[b, pl.ds(base + i * _CHUNK, _CHUNK)],
                wsem,
            )
            for b in range(_B)
        ]
    for w in writes:
        w.wait()


def kernel(seq_in_embeds, embedding):
    del seq_in_embeds
    return _broadcast_rows(embedding)
